# baseline (speedup 1.0000x reference)
import functools

import jax
import jax.numpy as jnp
from jax import lax
from jax.experimental import pallas as pl
from jax.experimental.pallas import tpu as pltpu
from jax.experimental.pallas import tpu_sc as plsc

_LAMBDA_HM = 1.0
_LAMBDA_OFF = 1.0
_LAMBDA_FLUX = 0.1
_SIGMA = 2.0

_H = 256
_W = 256
_K = 64
_B = 8
_NC = 2
_NSUB = 16
_NW = _NC * _NSUB
_NSTRIP = _NW // _B
_SR = _H // _NSTRIP
_MAGIC = 12582912.0


def _sc_body(gtx_h, gty_h, gtf_h, off0_h, off1_h, flux_h,
             canvas_h, part_h,
             cxf_r, cyf_r, cxi_r, cyi_r, dx_r, dy_r, enc_r, ridx_r,
             gtfv_r, g0_r, g1_r, gf_r, canvas_r, orow_r, sem):
    wid = lax.axis_index("s") * _NC + lax.axis_index("c")
    b = wid // _NSTRIP
    strip = wid % _NSTRIP
    strip_lo = strip * _SR
    lanes = lax.iota(jnp.int32, 16)
    radius_i = int(3 * _SIGMA + 1)

    pltpu.sync_copy(gtx_h.at[b], cxf_r)
    pltpu.sync_copy(gty_h.at[b], cyf_r)
    pltpu.sync_copy(gtf_h.at[b], gtfv_r)

    for c in range(_K // 16):
        sl = pl.ds(c * 16, 16)
        cx = cxf_r[sl] * float(_W - 1)
        cy = cyf_r[sl] * float(_H - 1)
        rcx = jnp.minimum(jnp.maximum((cx + _MAGIC) - _MAGIC, 0.0),
                          float(_W - 1))
        rcy = jnp.minimum(jnp.maximum((cy + _MAGIC) - _MAGIC, 0.0),
                          float(_H - 1))
        cxi = rcx.astype(jnp.int32)
        cyi = rcy.astype(jnp.int32)
        cxf_r[sl] = cx
        cyf_r[sl] = cy
        cxi_r[sl] = cxi
        cyi_r[sl] = cyi
        dx_r[sl] = cx - rcx
        dy_r[sl] = cy - rcy
        enc = cyi * _W + cxi
        enc_r[sl] = enc
        ridx_r[sl] = enc >> 7

    zero16 = jnp.zeros((16,), jnp.float32)

    def zbody(i, _):
        for j in range(8):
            canvas_r[pl.ds(i * 128 + j * 16, 16)] = zero16
        return 0

    lax.fori_loop(0, _SR * _W // 128, zbody, 0)

    def kbody(k, _):
        cxik = cxi_r[pl.ds(k, 16)][0]
        cyik = cyi_r[pl.ds(k, 16)][0]
        cxfk = cxf_r[pl.ds(k, 16)][0]
        cyfk = cyf_r[pl.ds(k, 16)][0]
        r_lo = jnp.maximum(cyik - radius_i, strip_lo)
        r_hi = jnp.minimum(cyik + radius_i + 1, strip_lo + _SR)
        xvec = cxik - radius_i + lanes
        valid = (xvec >= 0) & (xvec <= _W - 1) & (lanes <= 2 * radius_i)
        xc = jnp.minimum(jnp.maximum(xvec, 0), _W - 1)
        xd = xvec.astype(jnp.float32) - cxfk
        dx2 = xd * xd

        def rbody(r, _):
            rf = (lanes * 0 + r).astype(jnp.float32)
            yd = rf - cyfk
            g = jnp.exp(-(dx2 + yd * yd) / (2.0 * _SIGMA ** 2))
            idx = jnp.where(valid, (r - strip_lo) * _W + xc,
                            _SR * _W + lanes)
            old = plsc.load_gather(canvas_r, [idx])
            plsc.store_scatter(canvas_r, [idx], jnp.maximum(old, g))
            return 0

        lax.fori_loop(r_lo, r_hi, rbody, 0)
        return 0

    lax.fori_loop(0, _K, kbody, 0)
    pltpu.sync_copy(canvas_r.at[pl.ds(0, _SR * _W)], canvas_h.at[b, strip])

    encv = [enc_r[pl.ds(c * 16, 16)] for c in range(_K // 16)]
    posv = [lanes + c * 16 for c in range(_K // 16)]

    def wbody(kp, dup):
        e = enc_r[pl.ds(kp, 16)][0]
        return tuple(
            dup[c] | ((encv[c] == e) & (posv[c] > kp)).astype(jnp.int32)
            for c in range(_K // 16))

    zi = jnp.zeros((16,), jnp.int32)
    dup = lax.fori_loop(0, _K, wbody, (zi, zi, zi, zi))

    pltpu.async_copy(off0_h.at[b].at[ridx_r], g0_r, sem).wait()
    pltpu.async_copy(off1_h.at[b].at[ridx_r], g1_r, sem).wait()
    pltpu.async_copy(flux_h.at[b].at[ridx_r], gf_r, sem).wait()

    a_off = jnp.zeros((16,), jnp.float32)
    a_flux = jnp.zeros((16,), jnp.float32)
    a_np = jnp.zeros((16,), jnp.float32)
    for c in range(_K // 16):
        sl = pl.ds(c * 16, 16)
        cyiv = cyi_r[sl]
        m = (dup[c] == 0) & (cyiv >= strip_lo) & (cyiv < strip_lo + _SR)
        kidx = lanes + c * 16
        rem = encv[c] & 127
        v0 = plsc.load_gather(g0_r, [kidx, rem])
        v1 = plsc.load_gather(g1_r, [kidx, rem])
        vf = plsc.load_gather(gf_r, [kidx, rem])
        contrib = jnp.abs(v0 - dx_r[sl]) + jnp.abs(v1 - dy_r[sl])
        a_off = a_off + jnp.where(m, contrib, 0.0)
        a_flux = a_flux + jnp.where(m, jnp.abs(vf - gtfv_r[sl]), 0.0)
        a_np = a_np + jnp.where(m, 1.0, 0.0)

    s_off = jnp.sum(a_off)
    s_flux = jnp.sum(a_flux)
    s_np = jnp.sum(a_np)
    orow_r[pl.ds(0, 16)] = (jnp.where(lanes == 0, s_off, 0.0)
                            + jnp.where(lanes == 1, s_flux, 0.0)
                            + jnp.where(lanes == 2, s_np, 0.0))
    for c in range(1, 8):
        orow_r[pl.ds(c * 16, 16)] = zero16
    pltpu.sync_copy(orow_r, part_h.at[wid])


_sc_render = functools.partial(
    pl.kernel,
    mesh=plsc.VectorSubcoreMesh(core_axis_name="c", subcore_axis_name="s"),
    compiler_params=pltpu.CompilerParams(needs_layout_passes=False),
    out_type=[
        jax.ShapeDtypeStruct((_B, _NSTRIP, _SR * _W), jnp.float32),
        jax.ShapeDtypeStruct((_NW, 128), jnp.float32),
    ],
    scratch_types=[
        pltpu.VMEM((128,), jnp.float32),
        pltpu.VMEM((128,), jnp.float32),
        pltpu.VMEM((_K + 16,), jnp.int32),
        pltpu.VMEM((_K + 16,), jnp.int32),
        pltpu.VMEM((_K,), jnp.float32),
        pltpu.VMEM((_K,), jnp.float32),
        pltpu.VMEM((_K + 16,), jnp.int32),
        pltpu.VMEM((_K,), jnp.int32),
        pltpu.VMEM((128,), jnp.float32),
        pltpu.VMEM((_K, 128), jnp.float32),
        pltpu.VMEM((_K, 128), jnp.float32),
        pltpu.VMEM((_K, 128), jnp.float32),
        pltpu.VMEM((_SR * _W + 16,), jnp.float32),
        pltpu.VMEM((128,), jnp.float32),
        pltpu.SemaphoreType.DMA,
    ],
)(_sc_body)


def _tc_focal_body(hm_ref, t_ref, out_ref):
    p = jnp.clip(hm_ref[...], 1e-6, 1.0 - 1e-6)
    t = t_ref[...]
    pos = t == 1.0
    one_m_p = 1.0 - p
    pos_l = -(one_m_p * one_m_p) * jnp.log(p)
    omt = 1.0 - t
    omt2 = omt * omt
    neg_l = -(omt2 * omt2) * (p * p) * jnp.log(1.0 - p)
    s_f = jnp.sum(jnp.where(pos, pos_l, neg_l))
    pc = jnp.sum(pos.astype(jnp.float32))
    lane = jax.lax.broadcasted_iota(jnp.int32, (1, 1, 128), 2)
    out_ref[...] = (jnp.where(lane == 0, s_f, 0.0)
                    + jnp.where(lane == 1, pc, 0.0))


@jax.jit
def _run(heatmap, offset, log_flux, gt_centroids, gt_log_flux):
    B = heatmap.shape[0]
    pad = ((0, 0), (0, 128 - _K))
    gtx = jnp.pad(gt_centroids[:, :, 0], pad)
    gty = jnp.pad(gt_centroids[:, :, 1], pad)
    gtf128 = jnp.pad(gt_log_flux, pad)
    off0 = offset[:, 0].reshape(B, _H * _W // 128, 128)
    off1 = offset[:, 1].reshape(B, _H * _W // 128, 128)
    fluxf = log_flux.reshape(B, _H * _W // 128, 128)

    canvas, part = _sc_render(gtx, gty, gtf128, off0, off1, fluxf)

    hm2 = heatmap.reshape(B * _H, _W)
    t2 = canvas.reshape(B * _H, _W)
    out = pl.pallas_call(
        _tc_focal_body,
        grid=(B,),
        in_specs=[
            pl.BlockSpec((_H, _W), lambda i: (i, 0)),
            pl.BlockSpec((_H, _W), lambda i: (i, 0)),
        ],
        out_specs=pl.BlockSpec((1, 1, 128), lambda i: (i, 0, 0)),
        out_shape=jax.ShapeDtypeStruct((B, 1, 128), jnp.float32),
    )(hm2, t2)

    s_f = jnp.sum(out[:, 0, 0])
    pc = jnp.sum(out[:, 0, 1])
    s_off = jnp.sum(part[:, 0])
    s_flux = jnp.sum(part[:, 1])
    n_pos = jnp.sum(part[:, 2])
    loss_hm = s_f / jnp.maximum(pc, 1.0)
    n_pos_c = jnp.maximum(n_pos, 1.0)
    l_hm = _LAMBDA_HM * loss_hm
    l_off = _LAMBDA_OFF * (s_off / n_pos_c)
    l_fl = _LAMBDA_FLUX * (s_flux / n_pos_c)
    total = l_hm + l_off + l_fl
    return l_hm, l_off, l_fl, total


def kernel(heatmap, offset, log_flux, gt_centroids, gt_log_flux):
    K = gt_centroids.shape[1]
    l_hm, l_off, l_fl, total = _run(heatmap, offset, log_flux,
                                    gt_centroids, gt_log_flux)
    return (l_hm, l_off, l_fl, total, jnp.asarray(float(K), jnp.float32))


# speedup vs baseline: 1.2401x; 1.2401x over previous
import functools

import jax
import jax.numpy as jnp
from jax import lax
from jax.experimental import pallas as pl
from jax.experimental.pallas import tpu as pltpu
from jax.experimental.pallas import tpu_sc as plsc

_LAMBDA_HM = 1.0
_LAMBDA_OFF = 1.0
_LAMBDA_FLUX = 0.1
_SIGMA = 2.0

_H = 256
_W = 256
_K = 64
_B = 8
_NC = 2
_NSUB = 16
_NW = _NC * _NSUB
_NSTRIP = _NW // _B
_SR = _H // _NSTRIP
_SW = _SR * _W
_MAGIC = 12582912.0


def _sc_body(gtc_h, gtf_h, off_h, flux_h,
             canvas_h, part_h,
             gtc_r, cxf_r, cyf_r, cxi_r, cyi_r, dx_r, dy_r, enc_r, ridx_r,
             gtfv_r, g0_r, g1_r, gf_r, canvas_r, orow_r, sem):
    wid = lax.axis_index("s") * _NC + lax.axis_index("c")
    b = wid // _NSTRIP
    strip = wid % _NSTRIP
    strip_lo = strip * _SR
    lanes = lax.iota(jnp.int32, 16)
    radius_i = int(3 * _SIGMA + 1)

    pltpu.sync_copy(gtc_h.at[b], gtc_r)
    pltpu.sync_copy(gtf_h.at[b], gtfv_r)

    for c in range(_K // 16):
        sl = pl.ds(c * 16, 16)
        kidx2 = (lanes + c * 16) * 2
        cx = plsc.load_gather(gtc_r, [kidx2]) * float(_W - 1)
        cy = plsc.load_gather(gtc_r, [kidx2 + 1]) * float(_H - 1)
        rcx = jnp.minimum(jnp.maximum((cx + _MAGIC) - _MAGIC, 0.0),
                          float(_W - 1))
        rcy = jnp.minimum(jnp.maximum((cy + _MAGIC) - _MAGIC, 0.0),
                          float(_H - 1))
        cxi = rcx.astype(jnp.int32)
        cyi = rcy.astype(jnp.int32)
        cxf_r[sl] = cx
        cyf_r[sl] = cy
        cxi_r[sl] = cxi
        cyi_r[sl] = cyi
        dx_r[sl] = cx - rcx
        dy_r[sl] = cy - rcy
        enc = cyi * _W + cxi
        enc_r[sl] = enc
        ridx_r[sl] = enc >> 7

    zero16 = jnp.zeros((16,), jnp.float32)

    def zbody(i, _):
        for j in range(8):
            canvas_r[pl.ds(i * 128 + j * 16, 16)] = zero16
        return 0

    lax.fori_loop(0, _SW // 128, zbody, 0)

    def kbody(k, _):
        cxik = cxi_r[pl.ds(k, 16)][0]
        cyik = cyi_r[pl.ds(k, 16)][0]
        cxfk = cxf_r[pl.ds(k, 16)][0]
        cyfk = cyf_r[pl.ds(k, 16)][0]
        r_lo = jnp.maximum(cyik - radius_i, strip_lo)
        r_hi = jnp.minimum(cyik + radius_i + 1, strip_lo + _SR)
        xvec = cxik - radius_i + lanes
        valid = (xvec >= 0) & (xvec <= _W - 1) & (lanes <= 2 * radius_i)
        xc = jnp.minimum(jnp.maximum(xvec, 0), _W - 1)
        xd = xvec.astype(jnp.float32) - cxfk
        dx2 = xd * xd

        def rbody(r, _):
            rf = (lanes * 0 + r).astype(jnp.float32)
            yd = rf - cyfk
            g = jnp.exp(-(dx2 + yd * yd) / (2.0 * _SIGMA ** 2))
            idx = jnp.where(valid, (r - strip_lo) * _W + xc, _SW + lanes)
            old = plsc.load_gather(canvas_r, [idx])
            plsc.store_scatter(canvas_r, [idx], jnp.maximum(old, g))
            return 0

        lax.fori_loop(r_lo, r_hi, rbody, 0)
        return 0

    lax.fori_loop(0, _K, kbody, 0)
    pltpu.sync_copy(canvas_r.at[pl.ds(0, _SW)],
                    canvas_h.at[pl.ds((b * _NSTRIP + strip) * _SW, _SW)])

    encv = [enc_r[pl.ds(c * 16, 16)] for c in range(_K // 16)]
    posv = [lanes + c * 16 for c in range(_K // 16)]

    def wbody(kp, dup):
        e = enc_r[pl.ds(kp, 16)][0]
        return tuple(
            dup[c] | ((encv[c] == e) & (posv[c] > kp)).astype(jnp.int32)
            for c in range(_K // 16))

    zi = jnp.zeros((16,), jnp.int32)
    dup = lax.fori_loop(0, _K, wbody, (zi, zi, zi, zi))

    pltpu.async_copy(off_h.at[b, 0].at[ridx_r], g0_r, sem).wait()
    pltpu.async_copy(off_h.at[b, 1].at[ridx_r], g1_r, sem).wait()
    pltpu.async_copy(flux_h.at[b].at[ridx_r], gf_r, sem).wait()

    a_off = jnp.zeros((16,), jnp.float32)
    a_flux = jnp.zeros((16,), jnp.float32)
    a_np = jnp.zeros((16,), jnp.float32)
    for c in range(_K // 16):
        sl = pl.ds(c * 16, 16)
        cyiv = cyi_r[sl]
        m = (dup[c] == 0) & (cyiv >= strip_lo) & (cyiv < strip_lo + _SR)
        kidx = lanes + c * 16
        rem = encv[c] & 127
        v0 = plsc.load_gather(g0_r, [kidx, rem])
        v1 = plsc.load_gather(g1_r, [kidx, rem])
        vf = plsc.load_gather(gf_r, [kidx, rem])
        contrib = jnp.abs(v0 - dx_r[sl]) + jnp.abs(v1 - dy_r[sl])
        a_off = a_off + jnp.where(m, contrib, 0.0)
        a_flux = a_flux + jnp.where(m, jnp.abs(vf - gtfv_r[sl]), 0.0)
        a_np = a_np + jnp.where(m, 1.0, 0.0)

    s_off = jnp.sum(a_off)
    s_flux = jnp.sum(a_flux)
    s_np = jnp.sum(a_np)
    orow_r[pl.ds(0, 16)] = (jnp.where(lanes == 0, s_off, 0.0)
                            + jnp.where(lanes == 1, s_flux, 0.0)
                            + jnp.where(lanes == 2, s_np, 0.0))
    for c in range(1, 8):
        orow_r[pl.ds(c * 16, 16)] = zero16
    pltpu.sync_copy(orow_r, part_h.at[wid])


_sc_render = functools.partial(
    pl.kernel,
    mesh=plsc.VectorSubcoreMesh(core_axis_name="c", subcore_axis_name="s"),
    compiler_params=pltpu.CompilerParams(needs_layout_passes=False),
    out_type=[
        jax.ShapeDtypeStruct((_B * _H * _W,), jnp.float32),
        jax.ShapeDtypeStruct((_NW, 128), jnp.float32),
    ],
    scratch_types=[
        pltpu.VMEM((128,), jnp.float32),
        pltpu.VMEM((_K + 16,), jnp.float32),
        pltpu.VMEM((_K + 16,), jnp.float32),
        pltpu.VMEM((_K + 16,), jnp.int32),
        pltpu.VMEM((_K + 16,), jnp.int32),
        pltpu.VMEM((_K,), jnp.float32),
        pltpu.VMEM((_K,), jnp.float32),
        pltpu.VMEM((_K + 16,), jnp.int32),
        pltpu.VMEM((_K,), jnp.int32),
        pltpu.VMEM((128,), jnp.float32),
        pltpu.VMEM((_K, 128), jnp.float32),
        pltpu.VMEM((_K, 128), jnp.float32),
        pltpu.VMEM((_K, 128), jnp.float32),
        pltpu.VMEM((_SW + 16,), jnp.float32),
        pltpu.VMEM((128,), jnp.float32),
        pltpu.SemaphoreType.DMA,
    ],
)(_sc_body)


def _tc_focal_body(part_ref, hm_ref, t_ref, out_ref):
    i = pl.program_id(0)
    p = jnp.clip(hm_ref[...], 1e-6, 1.0 - 1e-6)
    t = t_ref[...]
    pos = t == 1.0
    one_m_p = 1.0 - p
    pos_l = -(one_m_p * one_m_p) * jnp.log(p)
    omt = 1.0 - t
    omt2 = omt * omt
    neg_l = -(omt2 * omt2) * (p * p) * jnp.log(1.0 - p)
    s_f = jnp.sum(jnp.where(pos, pos_l, neg_l))
    pc = jnp.sum(pos.astype(jnp.float32))
    lane = jax.lax.broadcasted_iota(jnp.int32, (1, 128), 1)
    acc = jnp.where(lane == 0, s_f, 0.0) + jnp.where(lane == 1, pc, 0.0)

    @pl.when(i == 0)
    def _():
        out_ref[...] = jnp.zeros_like(out_ref)

    out_ref[...] = out_ref[...] + acc

    @pl.when(i == _B - 1)
    def _():
        cur = out_ref[...]
        s_f_t = jnp.sum(jnp.where(lane == 0, cur, 0.0))
        pc_t = jnp.sum(jnp.where(lane == 1, cur, 0.0))
        pmat = part_ref[...]
        lane2 = jax.lax.broadcasted_iota(jnp.int32, (_NW, 128), 1)
        s_off = jnp.sum(jnp.where(lane2 == 0, pmat, 0.0))
        s_flux = jnp.sum(jnp.where(lane2 == 1, pmat, 0.0))
        n_pos = jnp.sum(jnp.where(lane2 == 2, pmat, 0.0))
        loss_hm = s_f_t / jnp.maximum(pc_t, 1.0)
        n_pos_c = jnp.maximum(n_pos, 1.0)
        l_hm = _LAMBDA_HM * loss_hm
        l_off = _LAMBDA_OFF * (s_off / n_pos_c)
        l_fl = _LAMBDA_FLUX * (s_flux / n_pos_c)
        total = l_hm + l_off + l_fl
        out_ref[...] = (jnp.where(lane == 0, l_hm, 0.0)
                        + jnp.where(lane == 1, l_off, 0.0)
                        + jnp.where(lane == 2, l_fl, 0.0)
                        + jnp.where(lane == 3, total, 0.0)
                        + jnp.where(lane == 4, float(_K), 0.0))


@jax.jit
def _run(heatmap, offset, log_flux, gt_centroids, gt_log_flux):
    B = heatmap.shape[0]
    gtc128 = gt_centroids.reshape(B, 2 * _K)
    gtf128 = jnp.pad(gt_log_flux, ((0, 0), (0, 128 - _K)))
    off4 = offset.reshape(B, 2, _H * _W // 128, 128)
    flux3 = log_flux.reshape(B, _H * _W // 128, 128)

    canvas, part = _sc_render(gtc128, gtf128, off4, flux3)

    hm2 = heatmap.reshape(B * _H, _W)
    t2 = canvas.reshape(B * _H, _W)
    out = pl.pallas_call(
        _tc_focal_body,
        grid=(B,),
        in_specs=[
            pl.BlockSpec((_NW, 128), lambda i: (0, 0)),
            pl.BlockSpec((_H, _W), lambda i: (i, 0)),
            pl.BlockSpec((_H, _W), lambda i: (i, 0)),
        ],
        out_specs=pl.BlockSpec((1, 128), lambda i: (0, 0)),
        out_shape=jax.ShapeDtypeStruct((1, 128), jnp.float32),
    )(part, hm2, t2)
    return out[0, 0], out[0, 1], out[0, 2], out[0, 3]


def kernel(heatmap, offset, log_flux, gt_centroids, gt_log_flux):
    K = gt_centroids.shape[1]
    l_hm, l_off, l_fl, total = _run(heatmap, offset, log_flux,
                                    gt_centroids, gt_log_flux)
    return (l_hm, l_off, l_fl, total, jnp.asarray(float(K), jnp.float32))


# speedup vs baseline: 1.4152x; 1.1412x over previous
import functools

import jax
import jax.numpy as jnp
from jax import lax
from jax.experimental import pallas as pl
from jax.experimental.pallas import tpu as pltpu
from jax.experimental.pallas import tpu_sc as plsc

_LAMBDA_HM = 1.0
_LAMBDA_OFF = 1.0
_LAMBDA_FLUX = 0.1
_SIGMA = 2.0

_H = 256
_W = 256
_K = 64
_B = 8
_NC = 2
_NSUB = 16
_NW = _NC * _NSUB
_NSTRIP = _NW // _B
_SR = _H // _NSTRIP
_SW = _SR * _W
_MAGIC = 12582912.0


def _sc_body(gtc_h, gtf_h, off_h, flux_h,
             canvas_h, part_h,
             gtc_r, cxf_r, cyf_r, cxi_r, cyi_r, dx_r, dy_r, enc_r, ridx_r,
             gtfv_r, g0_r, g1_r, gf_r, canvas_r, orow_r, sem):
    wid = lax.axis_index("s") * _NC + lax.axis_index("c")
    b = wid // _NSTRIP
    strip = wid % _NSTRIP
    strip_lo = strip * _SR
    lanes = lax.iota(jnp.int32, 16)
    radius_i = int(3 * _SIGMA + 1)

    pltpu.sync_copy(gtc_h.at[b], gtc_r)
    pltpu.sync_copy(gtf_h.at[b], gtfv_r)

    for c in range(_K // 16):
        sl = pl.ds(c * 16, 16)
        kidx2 = (lanes + c * 16) * 2
        cx = plsc.load_gather(gtc_r, [kidx2]) * float(_W - 1)
        cy = plsc.load_gather(gtc_r, [kidx2 + 1]) * float(_H - 1)
        rcx = jnp.minimum(jnp.maximum((cx + _MAGIC) - _MAGIC, 0.0),
                          float(_W - 1))
        rcy = jnp.minimum(jnp.maximum((cy + _MAGIC) - _MAGIC, 0.0),
                          float(_H - 1))
        cxi = rcx.astype(jnp.int32)
        cyi = rcy.astype(jnp.int32)
        cxf_r[sl] = cx
        cyf_r[sl] = cy
        cxi_r[sl] = cxi
        cyi_r[sl] = cyi
        dx_r[sl] = cx - rcx
        dy_r[sl] = cy - rcy
        enc_r[sl] = cyi * _W + cxi
        ridx_r[sl] = cyi

    zero16 = jnp.zeros((16,), jnp.float32)

    def zbody(i, _):
        for j in range(16):
            canvas_r[i, pl.ds(j * 16, 16)] = zero16
        return 0

    lax.fori_loop(0, _SR, zbody, 0)

    def kbody(k, _):
        cxik = cxi_r[pl.ds(k, 16)][0]
        cyik = cyi_r[pl.ds(k, 16)][0]
        cxfk = cxf_r[pl.ds(k, 16)][0]
        cyfk = cyf_r[pl.ds(k, 16)][0]
        r_lo = jnp.maximum(cyik - radius_i, strip_lo)
        r_hi = jnp.minimum(cyik + radius_i + 1, strip_lo + _SR)
        xvec = cxik - radius_i + lanes
        valid = (xvec >= 0) & (xvec <= _W - 1) & (lanes <= 2 * radius_i)
        xc = jnp.minimum(jnp.maximum(xvec, 0), _W - 1)
        xd = xvec.astype(jnp.float32) - cxfk
        dx2 = xd * xd

        def rbody(r, _):
            rf = (lanes * 0 + r).astype(jnp.float32)
            yd = rf - cyfk
            g = jnp.exp(-(dx2 + yd * yd) / (2.0 * _SIGMA ** 2))
            rowv = jnp.where(valid, lanes * 0 + (r - strip_lo), _SR)
            colv = jnp.where(valid, xc, lanes)
            old = plsc.load_gather(canvas_r, [rowv, colv])
            plsc.store_scatter(canvas_r, [rowv, colv], jnp.maximum(old, g))
            return 0

        lax.fori_loop(r_lo, r_hi, rbody, 0)
        return 0

    lax.fori_loop(0, _K, kbody, 0)
    pltpu.sync_copy(canvas_r.at[pl.ds(0, _SR)],
                    canvas_h.at[pl.ds((b * _NSTRIP + strip) * _SR, _SR)])

    encv = [enc_r[pl.ds(c * 16, 16)] for c in range(_K // 16)]
    posv = [lanes + c * 16 for c in range(_K // 16)]

    def wbody(kp, dup):
        e = enc_r[pl.ds(kp, 16)][0]
        return tuple(
            dup[c] | ((encv[c] == e) & (posv[c] > kp)).astype(jnp.int32)
            for c in range(_K // 16))

    zi = jnp.zeros((16,), jnp.int32)
    dup = lax.fori_loop(0, _K, wbody, (zi, zi, zi, zi))

    pltpu.async_copy(off_h.at[b, 0].at[ridx_r], g0_r, sem).wait()
    pltpu.async_copy(off_h.at[b, 1].at[ridx_r], g1_r, sem).wait()
    pltpu.async_copy(flux_h.at[b].at[ridx_r], gf_r, sem).wait()

    a_off = jnp.zeros((16,), jnp.float32)
    a_flux = jnp.zeros((16,), jnp.float32)
    a_np = jnp.zeros((16,), jnp.float32)
    for c in range(_K // 16):
        sl = pl.ds(c * 16, 16)
        cyiv = cyi_r[sl]
        m = (dup[c] == 0) & (cyiv >= strip_lo) & (cyiv < strip_lo + _SR)
        kidx = lanes + c * 16
        rem = encv[c] & (_W - 1)
        v0 = plsc.load_gather(g0_r, [kidx, rem])
        v1 = plsc.load_gather(g1_r, [kidx, rem])
        vf = plsc.load_gather(gf_r, [kidx, rem])
        contrib = jnp.abs(v0 - dx_r[sl]) + jnp.abs(v1 - dy_r[sl])
        a_off = a_off + jnp.where(m, contrib, 0.0)
        a_flux = a_flux + jnp.where(m, jnp.abs(vf - gtfv_r[sl]), 0.0)
        a_np = a_np + jnp.where(m, 1.0, 0.0)

    s_off = jnp.sum(a_off)
    s_flux = jnp.sum(a_flux)
    s_np = jnp.sum(a_np)
    orow_r[pl.ds(0, 16)] = (jnp.where(lanes == 0, s_off, 0.0)
                            + jnp.where(lanes == 1, s_flux, 0.0)
                            + jnp.where(lanes == 2, s_np, 0.0))
    for c in range(1, 8):
        orow_r[pl.ds(c * 16, 16)] = zero16
    pltpu.sync_copy(orow_r, part_h.at[wid])


_sc_render = functools.partial(
    pl.kernel,
    mesh=plsc.VectorSubcoreMesh(core_axis_name="c", subcore_axis_name="s"),
    compiler_params=pltpu.CompilerParams(needs_layout_passes=False),
    out_type=[
        jax.ShapeDtypeStruct((_B * _H, _W), jnp.float32),
        jax.ShapeDtypeStruct((_NW, 128), jnp.float32),
    ],
    scratch_types=[
        pltpu.VMEM((128,), jnp.float32),
        pltpu.VMEM((_K + 16,), jnp.float32),
        pltpu.VMEM((_K + 16,), jnp.float32),
        pltpu.VMEM((_K + 16,), jnp.int32),
        pltpu.VMEM((_K + 16,), jnp.int32),
        pltpu.VMEM((_K,), jnp.float32),
        pltpu.VMEM((_K,), jnp.float32),
        pltpu.VMEM((_K + 16,), jnp.int32),
        pltpu.VMEM((_K,), jnp.int32),
        pltpu.VMEM((128,), jnp.float32),
        pltpu.VMEM((_K, _W), jnp.float32),
        pltpu.VMEM((_K, _W), jnp.float32),
        pltpu.VMEM((_K, _W), jnp.float32),
        pltpu.VMEM((_SR + 1, _W), jnp.float32),
        pltpu.VMEM((128,), jnp.float32),
        pltpu.SemaphoreType.DMA,
    ],
)(_sc_body)


def _tc_focal_body(part_ref, hm_ref, t_ref, out_ref):
    i = pl.program_id(0)
    p = jnp.clip(hm_ref[...], 1e-6, 1.0 - 1e-6)
    t = t_ref[...]
    pos = t == 1.0
    one_m_p = 1.0 - p
    pos_l = -(one_m_p * one_m_p) * jnp.log(p)
    omt = 1.0 - t
    omt2 = omt * omt
    neg_l = -(omt2 * omt2) * (p * p) * jnp.log(1.0 - p)
    s_f = jnp.sum(jnp.where(pos, pos_l, neg_l))
    pc = jnp.sum(pos.astype(jnp.float32))
    lane = jax.lax.broadcasted_iota(jnp.int32, (1, 128), 1)
    acc = jnp.where(lane == 0, s_f, 0.0) + jnp.where(lane == 1, pc, 0.0)

    @pl.when(i == 0)
    def _():
        out_ref[...] = jnp.zeros_like(out_ref)

    out_ref[...] = out_ref[...] + acc

    @pl.when(i == _B - 1)
    def _():
        cur = out_ref[...]
        s_f_t = jnp.sum(jnp.where(lane == 0, cur, 0.0))
        pc_t = jnp.sum(jnp.where(lane == 1, cur, 0.0))
        pmat = part_ref[...]
        lane2 = jax.lax.broadcasted_iota(jnp.int32, (_NW, 128), 1)
        s_off = jnp.sum(jnp.where(lane2 == 0, pmat, 0.0))
        s_flux = jnp.sum(jnp.where(lane2 == 1, pmat, 0.0))
        n_pos = jnp.sum(jnp.where(lane2 == 2, pmat, 0.0))
        loss_hm = s_f_t / jnp.maximum(pc_t, 1.0)
        n_pos_c = jnp.maximum(n_pos, 1.0)
        l_hm = _LAMBDA_HM * loss_hm
        l_off = _LAMBDA_OFF * (s_off / n_pos_c)
        l_fl = _LAMBDA_FLUX * (s_flux / n_pos_c)
        total = l_hm + l_off + l_fl
        out_ref[...] = (jnp.where(lane == 0, l_hm, 0.0)
                        + jnp.where(lane == 1, l_off, 0.0)
                        + jnp.where(lane == 2, l_fl, 0.0)
                        + jnp.where(lane == 3, total, 0.0)
                        + jnp.where(lane == 4, float(_K), 0.0))


@jax.jit
def _run(heatmap, offset, log_flux, gt_centroids, gt_log_flux):
    B = heatmap.shape[0]
    gtc128 = gt_centroids.reshape(B, 2 * _K)
    gtf128 = jnp.pad(gt_log_flux, ((0, 0), (0, 128 - _K)))

    canvas, part = _sc_render(gtc128, gtf128, offset, log_flux)

    hm2 = heatmap.reshape(B * _H, _W)
    t2 = canvas
    out = pl.pallas_call(
        _tc_focal_body,
        grid=(B,),
        in_specs=[
            pl.BlockSpec((_NW, 128), lambda i: (0, 0)),
            pl.BlockSpec((_H, _W), lambda i: (i, 0)),
            pl.BlockSpec((_H, _W), lambda i: (i, 0)),
        ],
        out_specs=pl.BlockSpec((1, 128), lambda i: (0, 0)),
        out_shape=jax.ShapeDtypeStruct((1, 128), jnp.float32),
    )(part, hm2, t2)
    return out[0, 0], out[0, 1], out[0, 2], out[0, 3]


def kernel(heatmap, offset, log_flux, gt_centroids, gt_log_flux):
    K = gt_centroids.shape[1]
    l_hm, l_off, l_fl, total = _run(heatmap, offset, log_flux,
                                    gt_centroids, gt_log_flux)
    return (l_hm, l_off, l_fl, total, jnp.asarray(float(K), jnp.float32))
